# baseline (speedup 1.0000x reference)
import functools
import jax
import jax.numpy as jnp
from jax import lax
from jax.experimental import pallas as pl
from jax.experimental.pallas import tpu as pltpu, tpu_sc as plsc

N = 10000
D = 128
H = 128
NW = 32
NS = 16
LANES = 128
NB = 10240
SR = NB // NS
RB = 1000


def _mesh():
    return plsc.VectorSubcoreMesh(core_axis_name="c", subcore_axis_name="s")


def _deg_body(ch, dst2d, out, dstbuf, hist, accv, sbuf, shared):
    cid = lax.axis_index("c")
    sid = lax.axis_index("s")
    wid = sid * 2 + cid
    zeros = jnp.zeros((16,), jnp.float32)
    ones = jnp.ones((16,), jnp.float32)

    def zero_body(i, carry):
        hist[pl.ds(i * 16, 16)] = zeros
        return carry
    lax.fori_loop(0, NB // 16, zero_body, 0)

    pltpu.sync_copy(dst2d.at[pl.ds(wid * ch, ch)], dstbuf)

    def hist_body(j, carry):
        for k in range(LANES // 16):
            idx = dstbuf[j, pl.ds(k * 16, 16)]
            plsc.addupdate_scatter(hist, [idx], ones)
        return carry
    lax.fori_loop(0, ch, hist_body, 0)

    pltpu.sync_copy(hist, shared.at[sid])
    plsc.subcore_barrier()
    pltpu.sync_copy(shared.at[:, pl.ds(sid * SR, SR)], sbuf)

    def sum_body(c, carry):
        v = sbuf[0, pl.ds(c * 16, 16)]
        for r in range(1, NS):
            v = v + sbuf[r, pl.ds(c * 16, 16)]
        accv[pl.ds(c * 16, 16)] = v
        return carry
    lax.fori_loop(0, SR // 16, sum_body, 0)

    pltpu.sync_copy(accv, out.at[cid, pl.ds(sid * SR, SR)])


def _deg_hist(dst2d, ch):
    kfn = pl.kernel(
        functools.partial(_deg_body, ch),
        out_type=jax.ShapeDtypeStruct((2, NB), jnp.float32),
        mesh=_mesh(),
        scratch_types=[
            pltpu.VMEM((ch, LANES), jnp.int32),
            pltpu.VMEM((NB,), jnp.float32),
            pltpu.VMEM((SR,), jnp.float32),
            pltpu.VMEM((NS, SR), jnp.float32),
            pltpu.VMEM_SHARED((NS, NB), jnp.float32),
        ],
        compiler_params=pltpu.CompilerParams(needs_layout_passes=False),
    )
    return kfn(dst2d)


def _agg_kernel(ch, wd, src2d, dst2d, y):
    cpt = 2 * ch

    def body(src2d, dst2d, y, out, srcbuf, dstbuf, gbuf0, gbuf1, zbuf, acc,
             sem0, sem1):
        cid = lax.axis_index("c")
        sid = lax.axis_index("s")

        @pl.when(cid == 0)
        def _():
            zeros = jnp.zeros((16,), jnp.float32)

            def zrow(r, carry):
                for c in range(wd // 16):
                    zbuf[r, pl.ds(c * 16, 16)] = zeros
                return carry
            lax.fori_loop(0, 64, zrow, 0)

            def zstripe(t, carry):
                pltpu.sync_copy(zbuf, acc.at[pl.ds(sid * SR + t * 64, 64)])
                return carry
            lax.fori_loop(0, SR // 64, zstripe, 0)
            plsc.subcore_barrier()

            nseg = cpt // 8

            def seg_body(g, carry):
                base = sid * cpt + g * 8
                pltpu.sync_copy(src2d.at[pl.ds(base, 8)], srcbuf)
                pltpu.sync_copy(dst2d.at[pl.ds(base, 8)], dstbuf)
                pltpu.async_copy(y.at[srcbuf.at[0]], gbuf0, sem0)
                for j2 in range(4):
                    b = 2 * j2
                    pltpu.async_copy(y.at[srcbuf.at[b + 1]], gbuf1, sem1)
                    pltpu.make_async_copy(
                        y.at[srcbuf.at[b]], gbuf0, sem0).wait()
                    pltpu.sync_copy(gbuf0, acc.at[dstbuf.at[b]], add=True)
                    if b + 2 < 8:
                        pltpu.async_copy(y.at[srcbuf.at[b + 2]], gbuf0, sem0)
                    pltpu.make_async_copy(
                        y.at[srcbuf.at[b + 1]], gbuf1, sem1).wait()
                    pltpu.sync_copy(gbuf1, acc.at[dstbuf.at[b + 1]], add=True)
                return carry
            lax.fori_loop(0, nseg, seg_body, 0)

            plsc.subcore_barrier()
            pltpu.sync_copy(acc.at[pl.ds(sid * SR, SR)],
                            out.at[pl.ds(sid * SR, SR)])

    kfn = pl.kernel(
        body,
        out_type=jax.ShapeDtypeStruct((NB, wd), jnp.float32),
        mesh=_mesh(),
        scratch_types=[
            pltpu.VMEM((8, LANES), jnp.int32),
            pltpu.VMEM((8, LANES), jnp.int32),
            pltpu.VMEM((LANES, wd), jnp.float32),
            pltpu.VMEM((LANES, wd), jnp.float32),
            pltpu.VMEM((64, wd), jnp.float32),
            pltpu.VMEM_SHARED((NB, wd), jnp.float32),
            pltpu.SemaphoreType.DMA,
            pltpu.SemaphoreType.DMA,
        ],
    )
    return kfn(src2d, dst2d, y)


def _tc1_body(x, w1, degp, y, xw, dis):
    mm = jnp.dot(x[...], w1[...], preferred_element_type=jnp.float32)
    deg = degp[0] + degp[1] + 1.0
    di = lax.rsqrt(deg)
    xw[...] = mm
    y[...] = mm * di
    dis[...] = di


def _tc1(x, w1, degp):
    grid = (N // RB,)
    return pl.pallas_call(
        _tc1_body,
        grid=grid,
        in_specs=[
            pl.BlockSpec((RB, D), lambda i: (i, 0)),
            pl.BlockSpec((D, H), lambda i: (0, 0)),
            pl.BlockSpec((2, RB, 1), lambda i: (0, i, 0)),
        ],
        out_specs=[
            pl.BlockSpec((RB, H), lambda i: (i, 0)),
            pl.BlockSpec((RB, H), lambda i: (i, 0)),
            pl.BlockSpec((RB, 1), lambda i: (i, 0)),
        ],
        out_shape=[
            jax.ShapeDtypeStruct((N, H), jnp.float32),
            jax.ShapeDtypeStruct((N, H), jnp.float32),
            jax.ShapeDtypeStruct((N, 1), jnp.float32),
        ],
    )(x, w1, degp)


def _tc2_body(zp, xw, dis, b1, w2p, y2):
    di = dis[...]
    h = jnp.maximum(zp[...] * di + xw[...] * (di * di) + b1[...], 0.0)
    mm = jnp.dot(h, w2p[...], preferred_element_type=jnp.float32)
    y2[...] = mm * di


def _tc2(zp, xw, dis, b1, w2p, wo):
    grid = (N // RB,)
    return pl.pallas_call(
        _tc2_body,
        grid=grid,
        in_specs=[
            pl.BlockSpec((RB, H), lambda i: (i, 0)),
            pl.BlockSpec((RB, H), lambda i: (i, 0)),
            pl.BlockSpec((RB, 1), lambda i: (i, 0)),
            pl.BlockSpec((1, H), lambda i: (0, 0)),
            pl.BlockSpec((H, wo), lambda i: (0, 0)),
        ],
        out_specs=pl.BlockSpec((RB, wo), lambda i: (i, 0)),
        out_shape=jax.ShapeDtypeStruct((N, wo), jnp.float32),
    )(zp, xw, dis, b1, w2p)


def _agg2_kernel(ch2, src2d, dst2d, y2c0, y2c1, disp, b2p):
    def body(src2d, dst2d, y2c0, y2c1, disp, b2p, out,
             srcbuf, dstbuf, ybuf0, ybuf1, acc0, acc1, dbuf, bbuf,
             sbuf, obuf, shared):
        sid = lax.axis_index("s")

        pltpu.sync_copy(src2d.at[pl.ds(sid * ch2, ch2)], srcbuf)
        pltpu.sync_copy(dst2d.at[pl.ds(sid * ch2, ch2)], dstbuf)
        pltpu.sync_copy(y2c0, ybuf0)
        pltpu.sync_copy(y2c1, ybuf1)

        zeros = jnp.zeros((16,), jnp.float32)

        def zero_body(i, carry):
            acc0[pl.ds(i * 16, 16)] = zeros
            acc1[pl.ds(i * 16, 16)] = zeros
            return carry
        lax.fori_loop(0, NB // 16, zero_body, 0)

        @pl.when(sid == 0)
        def _():
            def seed_body(i, carry):
                acc0[pl.ds(i * 16, 16)] = ybuf0[pl.ds(i * 16, 16)]
                acc1[pl.ds(i * 16, 16)] = ybuf1[pl.ds(i * 16, 16)]
                return carry
            lax.fori_loop(0, N // 16, seed_body, 0)

        def edge_body(j, carry):
            for k in range(LANES // 16):
                s = srcbuf[j, pl.ds(k * 16, 16)]
                d = dstbuf[j, pl.ds(k * 16, 16)]
                v0 = plsc.load_gather(ybuf0, [s])
                v1 = plsc.load_gather(ybuf1, [s])
                plsc.addupdate_scatter(acc0, [d], v0)
                plsc.addupdate_scatter(acc1, [d], v1)
            return carry
        lax.fori_loop(0, ch2, edge_body, 0)

        pltpu.sync_copy(acc0, shared.at[sid, 0])
        pltpu.sync_copy(acc1, shared.at[sid, 1])
        plsc.subcore_barrier()

        pltpu.sync_copy(disp.at[pl.ds(sid * SR, SR)], dbuf)
        pltpu.sync_copy(b2p, bbuf)
        zi = jnp.zeros((16,), jnp.int32)
        b2_0 = plsc.load_gather(bbuf, [zi])
        b2_1 = plsc.load_gather(bbuf, [zi + 1])

        for col, bias in ((0, b2_0), (1, b2_1)):
            pltpu.sync_copy(shared.at[:, col, pl.ds(sid * SR, SR)], sbuf)

            def sum_body(c, carry):
                v = sbuf[0, pl.ds(c * 16, 16)]
                for r in range(1, NS):
                    v = v + sbuf[r, pl.ds(c * 16, 16)]
                obuf[pl.ds(c * 16, 16)] = v * dbuf[pl.ds(c * 16, 16)] + bias
                return carry
            lax.fori_loop(0, SR // 16, sum_body, 0)
            pltpu.sync_copy(obuf, out.at[col, pl.ds(sid * SR, SR)])

    kfn = pl.kernel(
        body,
        out_type=jax.ShapeDtypeStruct((2, NB), jnp.float32),
        mesh=plsc.VectorSubcoreMesh(
            core_axis_name="c", subcore_axis_name="s", num_cores=1),
        scratch_types=[
            pltpu.VMEM((ch2, LANES), jnp.int32),
            pltpu.VMEM((ch2, LANES), jnp.int32),
            pltpu.VMEM((N, ), jnp.float32),
            pltpu.VMEM((N, ), jnp.float32),
            pltpu.VMEM((NB,), jnp.float32),
            pltpu.VMEM((NB,), jnp.float32),
            pltpu.VMEM((SR,), jnp.float32),
            pltpu.VMEM((16,), jnp.float32),
            pltpu.VMEM((NS, SR), jnp.float32),
            pltpu.VMEM((SR,), jnp.float32),
            pltpu.VMEM_SHARED((NS, 2, NB), jnp.float32),
        ],
        compiler_params=pltpu.CompilerParams(needs_layout_passes=False),
    )
    return kfn(src2d, dst2d, y2c0, y2c1, disp, b2p)


@jax.jit
def kernel(x, edge_index, W1, b1, W2, b2):
    E = edge_index.shape[1]
    out_dim = W2.shape[1]
    WO = 16

    ch = -(-E // (NW * LANES))
    ch = ((ch + 7) // 8) * 8
    e_pad = NW * ch * LANES
    src = edge_index[0].astype(jnp.int32)
    dst = edge_index[1].astype(jnp.int32)
    pad = e_pad - E
    src = jnp.concatenate([src, jnp.zeros((pad,), jnp.int32)])
    dst = jnp.concatenate([dst, jnp.full((pad,), N, jnp.int32)])
    src2d = src.reshape(NW * ch, LANES)
    dst2d = dst.reshape(NW * ch, LANES)

    degp = _deg_hist(dst2d, ch)
    y, xw, dis = _tc1(x, W1, degp.reshape(2, NB, 1))
    zp = _agg_kernel(ch, H, src2d, dst2d, y)
    w2p = jnp.zeros((H, WO), jnp.float32).at[:, :out_dim].set(W2)
    y2 = _tc2(zp, xw, dis, b1.reshape(1, H), w2p, WO)
    y2c0 = y2[:, 0].reshape(N)
    y2c1 = y2[:, 1].reshape(N)
    disp = jnp.zeros((NB,), jnp.float32).at[:N].set(dis[:, 0])
    b2p = jnp.zeros((16,), jnp.float32).at[:out_dim].set(b2)
    ch2 = NW * ch // NS
    outT = _agg2_kernel(ch2, src2d, dst2d, y2c0, y2c1, disp, b2p)
    return outT[:, :N].T


# speedup vs baseline: 1.5070x; 1.5070x over previous
import functools
import jax
import jax.numpy as jnp
from jax import lax
from jax.experimental import pallas as pl
from jax.experimental.pallas import tpu as pltpu, tpu_sc as plsc

N = 10000
D = 128
H = 128
NW = 32
NS = 16
LANES = 128
NB = 10240
SR = NB // NS
RB = 1000


def _mesh():
    return plsc.VectorSubcoreMesh(core_axis_name="c", subcore_axis_name="s")


def _deg_body(ch, dst2d, out, dstbuf, hist, accv, sbuf, shared):
    cid = lax.axis_index("c")
    sid = lax.axis_index("s")
    wid = sid * 2 + cid
    zeros = jnp.zeros((16,), jnp.float32)
    ones = jnp.ones((16,), jnp.float32)

    def zero_body(i, carry):
        hist[pl.ds(i * 16, 16)] = zeros
        return carry
    lax.fori_loop(0, NB // 16, zero_body, 0)

    pltpu.sync_copy(dst2d.at[pl.ds(wid * ch, ch)], dstbuf)

    def hist_body(j, carry):
        for k in range(LANES // 16):
            idx = dstbuf[j, pl.ds(k * 16, 16)]
            plsc.addupdate_scatter(hist, [idx], ones)
        return carry
    lax.fori_loop(0, ch, hist_body, 0)

    pltpu.sync_copy(hist, shared.at[sid])
    plsc.subcore_barrier()
    pltpu.sync_copy(shared.at[:, pl.ds(sid * SR, SR)], sbuf)

    def sum_body(c, carry):
        v = sbuf[0, pl.ds(c * 16, 16)]
        for r in range(1, NS):
            v = v + sbuf[r, pl.ds(c * 16, 16)]
        accv[pl.ds(c * 16, 16)] = v
        return carry
    lax.fori_loop(0, SR // 16, sum_body, 0)

    pltpu.sync_copy(accv, out.at[cid, pl.ds(sid * SR, SR)])


def _deg_hist(dst2d, ch):
    kfn = pl.kernel(
        functools.partial(_deg_body, ch),
        out_type=jax.ShapeDtypeStruct((2, NB), jnp.float32),
        mesh=_mesh(),
        scratch_types=[
            pltpu.VMEM((ch, LANES), jnp.int32),
            pltpu.VMEM((NB,), jnp.float32),
            pltpu.VMEM((SR,), jnp.float32),
            pltpu.VMEM((NS, SR), jnp.float32),
            pltpu.VMEM_SHARED((NS, NB), jnp.float32),
        ],
        compiler_params=pltpu.CompilerParams(needs_layout_passes=False),
    )
    return kfn(dst2d)


def _agg_kernel(ch, wd, src2d, dst2d, y):
    ch0 = 144
    ch1 = 2 * ch - ch0

    def body(src2d, dst2d, y, out0, out1, srcbuf, dstbuf, gbuf0, gbuf1, zbuf,
             acc, sem0, sem1):
        cid = lax.axis_index("c")
        sid = lax.axis_index("s")

        zeros = jnp.zeros((16,), jnp.float32)

        def zrow(r, carry):
            for c in range(wd // 16):
                zbuf[r, pl.ds(c * 16, 16)] = zeros
            return carry
        lax.fori_loop(0, 64, zrow, 0)

        def zstripe(t, carry):
            pltpu.sync_copy(zbuf, acc.at[pl.ds(sid * SR + t * 64, 64)])
            return carry
        lax.fori_loop(0, SR // 64, zstripe, 0)
        plsc.subcore_barrier()

        nseg = jnp.where(cid == 0, ch0 // 8, ch1 // 8)
        tbase = jnp.where(cid == 0, sid * ch0, NS * ch0 + sid * ch1)

        def seg_body(g, carry):
            base = tbase + g * 8
            pltpu.sync_copy(src2d.at[pl.ds(base, 8)], srcbuf)
            pltpu.sync_copy(dst2d.at[pl.ds(base, 8)], dstbuf)
            pltpu.async_copy(y.at[srcbuf.at[0]], gbuf0, sem0)
            for j2 in range(4):
                b = 2 * j2
                pltpu.async_copy(y.at[srcbuf.at[b + 1]], gbuf1, sem1)
                pltpu.make_async_copy(
                    y.at[srcbuf.at[b]], gbuf0, sem0).wait()
                pltpu.sync_copy(gbuf0, acc.at[dstbuf.at[b]], add=True)
                if b + 2 < 8:
                    pltpu.async_copy(y.at[srcbuf.at[b + 2]], gbuf0, sem0)
                pltpu.make_async_copy(
                    y.at[srcbuf.at[b + 1]], gbuf1, sem1).wait()
                pltpu.sync_copy(gbuf1, acc.at[dstbuf.at[b + 1]], add=True)
            return carry
        lax.fori_loop(0, nseg, seg_body, 0)

        plsc.subcore_barrier()

        @pl.when(cid == 0)
        def _():
            pltpu.sync_copy(acc.at[pl.ds(sid * SR, SR)],
                            out0.at[pl.ds(sid * SR, SR)])

        @pl.when(cid == 1)
        def _():
            pltpu.sync_copy(acc.at[pl.ds(sid * SR, SR)],
                            out1.at[pl.ds(sid * SR, SR)])

    kfn = pl.kernel(
        body,
        out_type=(jax.ShapeDtypeStruct((NB, wd), jnp.float32),
                  jax.ShapeDtypeStruct((NB, wd), jnp.float32)),
        mesh=_mesh(),
        scratch_types=[
            pltpu.VMEM((8, LANES), jnp.int32),
            pltpu.VMEM((8, LANES), jnp.int32),
            pltpu.VMEM((LANES, wd), jnp.float32),
            pltpu.VMEM((LANES, wd), jnp.float32),
            pltpu.VMEM((64, wd), jnp.float32),
            pltpu.VMEM_SHARED((NB, wd), jnp.float32),
            pltpu.SemaphoreType.DMA,
            pltpu.SemaphoreType.DMA,
        ],
    )
    return kfn(src2d, dst2d, y)


def _tc1_body(x, w1, degp, y, xw, dis):
    mm = jnp.dot(x[...], w1[...], preferred_element_type=jnp.float32)
    deg = degp[0] + degp[1] + 1.0
    di = lax.rsqrt(deg)
    xw[...] = mm
    y[...] = mm * di
    dis[...] = di


def _tc1(x, w1, degp):
    grid = (N // RB,)
    return pl.pallas_call(
        _tc1_body,
        grid=grid,
        in_specs=[
            pl.BlockSpec((RB, D), lambda i: (i, 0)),
            pl.BlockSpec((D, H), lambda i: (0, 0)),
            pl.BlockSpec((2, RB, 1), lambda i: (0, i, 0)),
        ],
        out_specs=[
            pl.BlockSpec((RB, H), lambda i: (i, 0)),
            pl.BlockSpec((RB, H), lambda i: (i, 0)),
            pl.BlockSpec((RB, 1), lambda i: (i, 0)),
        ],
        out_shape=[
            jax.ShapeDtypeStruct((N, H), jnp.float32),
            jax.ShapeDtypeStruct((N, H), jnp.float32),
            jax.ShapeDtypeStruct((N, 1), jnp.float32),
        ],
    )(x, w1, degp)


def _tc2_body(zp0, zp1, xw, dis, b1, w2p, y2):
    di = dis[...]
    z = zp0[...] + zp1[...]
    h = jnp.maximum(z * di + xw[...] * (di * di) + b1[...], 0.0)
    mm = jnp.dot(h, w2p[...], preferred_element_type=jnp.float32)
    y2[...] = mm * di


def _tc2(zp0, zp1, xw, dis, b1, w2p, wo):
    grid = (N // RB,)
    return pl.pallas_call(
        _tc2_body,
        grid=grid,
        in_specs=[
            pl.BlockSpec((RB, H), lambda i: (i, 0)),
            pl.BlockSpec((RB, H), lambda i: (i, 0)),
            pl.BlockSpec((RB, H), lambda i: (i, 0)),
            pl.BlockSpec((RB, 1), lambda i: (i, 0)),
            pl.BlockSpec((1, H), lambda i: (0, 0)),
            pl.BlockSpec((H, wo), lambda i: (0, 0)),
        ],
        out_specs=pl.BlockSpec((RB, wo), lambda i: (i, 0)),
        out_shape=jax.ShapeDtypeStruct((N, wo), jnp.float32),
    )(zp0, zp1, xw, dis, b1, w2p)


def _agg2_kernel(ch2, src2d, dst2d, y2c0, y2c1, disp, b2p):
    def body(src2d, dst2d, y2c0, y2c1, disp, b2p, out,
             srcbuf, dstbuf, ybuf0, ybuf1, acc0, acc1, dbuf, bbuf,
             sbuf, obuf, shared):
        sid = lax.axis_index("s")

        pltpu.sync_copy(src2d.at[pl.ds(sid * ch2, ch2)], srcbuf)
        pltpu.sync_copy(dst2d.at[pl.ds(sid * ch2, ch2)], dstbuf)
        pltpu.sync_copy(y2c0, ybuf0)
        pltpu.sync_copy(y2c1, ybuf1)

        zeros = jnp.zeros((16,), jnp.float32)

        def zero_body(i, carry):
            acc0[pl.ds(i * 16, 16)] = zeros
            acc1[pl.ds(i * 16, 16)] = zeros
            return carry
        lax.fori_loop(0, NB // 16, zero_body, 0)

        @pl.when(sid == 0)
        def _():
            def seed_body(i, carry):
                acc0[pl.ds(i * 16, 16)] = ybuf0[pl.ds(i * 16, 16)]
                acc1[pl.ds(i * 16, 16)] = ybuf1[pl.ds(i * 16, 16)]
                return carry
            lax.fori_loop(0, N // 16, seed_body, 0)

        def edge_body(j, carry):
            for k in range(LANES // 16):
                s = srcbuf[j, pl.ds(k * 16, 16)]
                d = dstbuf[j, pl.ds(k * 16, 16)]
                v0 = plsc.load_gather(ybuf0, [s])
                v1 = plsc.load_gather(ybuf1, [s])
                plsc.addupdate_scatter(acc0, [d], v0)
                plsc.addupdate_scatter(acc1, [d], v1)
            return carry
        lax.fori_loop(0, ch2, edge_body, 0)

        pltpu.sync_copy(acc0, shared.at[sid, 0])
        pltpu.sync_copy(acc1, shared.at[sid, 1])
        plsc.subcore_barrier()

        pltpu.sync_copy(disp.at[pl.ds(sid * SR, SR)], dbuf)
        pltpu.sync_copy(b2p, bbuf)
        zi = jnp.zeros((16,), jnp.int32)
        b2_0 = plsc.load_gather(bbuf, [zi])
        b2_1 = plsc.load_gather(bbuf, [zi + 1])

        for col, bias in ((0, b2_0), (1, b2_1)):
            pltpu.sync_copy(shared.at[:, col, pl.ds(sid * SR, SR)], sbuf)

            def sum_body(c, carry):
                v = sbuf[0, pl.ds(c * 16, 16)]
                for r in range(1, NS):
                    v = v + sbuf[r, pl.ds(c * 16, 16)]
                obuf[pl.ds(c * 16, 16)] = v * dbuf[pl.ds(c * 16, 16)] + bias
                return carry
            lax.fori_loop(0, SR // 16, sum_body, 0)
            pltpu.sync_copy(obuf, out.at[col, pl.ds(sid * SR, SR)])

    kfn = pl.kernel(
        body,
        out_type=jax.ShapeDtypeStruct((2, NB), jnp.float32),
        mesh=plsc.VectorSubcoreMesh(
            core_axis_name="c", subcore_axis_name="s", num_cores=1),
        scratch_types=[
            pltpu.VMEM((ch2, LANES), jnp.int32),
            pltpu.VMEM((ch2, LANES), jnp.int32),
            pltpu.VMEM((N, ), jnp.float32),
            pltpu.VMEM((N, ), jnp.float32),
            pltpu.VMEM((NB,), jnp.float32),
            pltpu.VMEM((NB,), jnp.float32),
            pltpu.VMEM((SR,), jnp.float32),
            pltpu.VMEM((16,), jnp.float32),
            pltpu.VMEM((NS, SR), jnp.float32),
            pltpu.VMEM((SR,), jnp.float32),
            pltpu.VMEM_SHARED((NS, 2, NB), jnp.float32),
        ],
        compiler_params=pltpu.CompilerParams(needs_layout_passes=False),
    )
    return kfn(src2d, dst2d, y2c0, y2c1, disp, b2p)


@jax.jit
def kernel(x, edge_index, W1, b1, W2, b2):
    E = edge_index.shape[1]
    out_dim = W2.shape[1]
    WO = 16

    ch = -(-E // (NW * LANES))
    ch = ((ch + 7) // 8) * 8
    e_pad = NW * ch * LANES
    src = edge_index[0].astype(jnp.int32)
    dst = edge_index[1].astype(jnp.int32)
    pad = e_pad - E
    src = jnp.concatenate([src, jnp.zeros((pad,), jnp.int32)])
    dst = jnp.concatenate([dst, jnp.full((pad,), N, jnp.int32)])
    src2d = src.reshape(NW * ch, LANES)
    dst2d = dst.reshape(NW * ch, LANES)

    degp = _deg_hist(dst2d, ch)
    y, xw, dis = _tc1(x, W1, degp.reshape(2, NB, 1))
    zp0, zp1 = _agg_kernel(ch, H, src2d, dst2d, y)
    w2p = jnp.zeros((H, WO), jnp.float32).at[:, :out_dim].set(W2)
    y2 = _tc2(zp0, zp1, xw, dis, b1.reshape(1, H), w2p, WO)
    y2c0 = y2[:, 0].reshape(N)
    y2c1 = y2[:, 1].reshape(N)
    disp = jnp.zeros((NB,), jnp.float32).at[:N].set(dis[:, 0])
    b2p = jnp.zeros((16,), jnp.float32).at[:out_dim].set(b2)
    ch2 = NW * ch // NS
    outT = _agg2_kernel(ch2, src2d, dst2d, y2c0, y2c1, disp, b2p)
    return outT[:, :N].T


# speedup vs baseline: 1.5165x; 1.0063x over previous
import functools
import jax
import jax.numpy as jnp
from jax import lax
from jax.experimental import pallas as pl
from jax.experimental.pallas import tpu as pltpu, tpu_sc as plsc

N = 10000
D = 128
H = 128
NW = 32
NS = 16
LANES = 128
NB = 10240
SR = NB // NS
RB = 1000


def _mesh():
    return plsc.VectorSubcoreMesh(core_axis_name="c", subcore_axis_name="s")


def _deg_body(ch, dst2d, out, dstbuf, hist, accv, sbuf, shared):
    cid = lax.axis_index("c")
    sid = lax.axis_index("s")
    wid = sid * 2 + cid
    zeros = jnp.zeros((16,), jnp.float32)
    ones = jnp.ones((16,), jnp.float32)

    def zero_body(i, carry):
        hist[pl.ds(i * 16, 16)] = zeros
        return carry
    lax.fori_loop(0, NB // 16, zero_body, 0)

    pltpu.sync_copy(dst2d.at[pl.ds(wid * ch, ch)], dstbuf)

    def hist_body(j, carry):
        for k in range(LANES // 16):
            idx = dstbuf[j, pl.ds(k * 16, 16)]
            plsc.addupdate_scatter(hist, [idx], ones)
        return carry
    lax.fori_loop(0, ch, hist_body, 0)

    pltpu.sync_copy(hist, shared.at[sid])
    plsc.subcore_barrier()
    pltpu.sync_copy(shared.at[:, pl.ds(sid * SR, SR)], sbuf)

    def sum_body(c, carry):
        v = sbuf[0, pl.ds(c * 16, 16)]
        for r in range(1, NS):
            v = v + sbuf[r, pl.ds(c * 16, 16)]
        accv[pl.ds(c * 16, 16)] = v
        return carry
    lax.fori_loop(0, SR // 16, sum_body, 0)

    pltpu.sync_copy(accv, out.at[cid, pl.ds(sid * SR, SR)])


def _deg_hist(dst2d, ch):
    kfn = pl.kernel(
        functools.partial(_deg_body, ch),
        out_type=jax.ShapeDtypeStruct((2, NB), jnp.float32),
        mesh=_mesh(),
        scratch_types=[
            pltpu.VMEM((ch, LANES), jnp.int32),
            pltpu.VMEM((NB,), jnp.float32),
            pltpu.VMEM((SR,), jnp.float32),
            pltpu.VMEM((NS, SR), jnp.float32),
            pltpu.VMEM_SHARED((NS, NB), jnp.float32),
        ],
        compiler_params=pltpu.CompilerParams(needs_layout_passes=False),
    )
    return kfn(dst2d)


def _agg_kernel(ch, wd, src2d, dst2d, y):
    ch0 = 152
    ch1 = 2 * ch - ch0

    def body(src2d, dst2d, y, out0, out1, srcbuf, dstbuf, gbuf0, gbuf1, zbuf,
             acc, sem0, sem1):
        cid = lax.axis_index("c")
        sid = lax.axis_index("s")

        zeros = jnp.zeros((16,), jnp.float32)

        def zrow(r, carry):
            for c in range(wd // 16):
                zbuf[r, pl.ds(c * 16, 16)] = zeros
            return carry
        lax.fori_loop(0, 64, zrow, 0)

        def zstripe(t, carry):
            pltpu.sync_copy(zbuf, acc.at[pl.ds(sid * SR + t * 64, 64)])
            return carry
        lax.fori_loop(0, SR // 64, zstripe, 0)
        plsc.subcore_barrier()

        nseg = jnp.where(cid == 0, ch0 // 8, ch1 // 8)
        tbase = jnp.where(cid == 0, sid * ch0, NS * ch0 + sid * ch1)

        def seg_body(g, carry):
            base = tbase + g * 8
            pltpu.sync_copy(src2d.at[pl.ds(base, 8)], srcbuf)
            pltpu.sync_copy(dst2d.at[pl.ds(base, 8)], dstbuf)
            pltpu.async_copy(y.at[srcbuf.at[0]], gbuf0, sem0)
            for j2 in range(4):
                b = 2 * j2
                pltpu.async_copy(y.at[srcbuf.at[b + 1]], gbuf1, sem1)
                pltpu.make_async_copy(
                    y.at[srcbuf.at[b]], gbuf0, sem0).wait()
                pltpu.sync_copy(gbuf0, acc.at[dstbuf.at[b]], add=True)
                if b + 2 < 8:
                    pltpu.async_copy(y.at[srcbuf.at[b + 2]], gbuf0, sem0)
                pltpu.make_async_copy(
                    y.at[srcbuf.at[b + 1]], gbuf1, sem1).wait()
                pltpu.sync_copy(gbuf1, acc.at[dstbuf.at[b + 1]], add=True)
            return carry
        lax.fori_loop(0, nseg, seg_body, 0)

        plsc.subcore_barrier()

        @pl.when(cid == 0)
        def _():
            pltpu.sync_copy(acc.at[pl.ds(sid * SR, SR)],
                            out0.at[pl.ds(sid * SR, SR)])

        @pl.when(cid == 1)
        def _():
            pltpu.sync_copy(acc.at[pl.ds(sid * SR, SR)],
                            out1.at[pl.ds(sid * SR, SR)])

    kfn = pl.kernel(
        body,
        out_type=(jax.ShapeDtypeStruct((NB, wd), jnp.float32),
                  jax.ShapeDtypeStruct((NB, wd), jnp.float32)),
        mesh=_mesh(),
        scratch_types=[
            pltpu.VMEM((8, LANES), jnp.int32),
            pltpu.VMEM((8, LANES), jnp.int32),
            pltpu.VMEM((LANES, wd), jnp.float32),
            pltpu.VMEM((LANES, wd), jnp.float32),
            pltpu.VMEM((64, wd), jnp.float32),
            pltpu.VMEM_SHARED((NB, wd), jnp.float32),
            pltpu.SemaphoreType.DMA,
            pltpu.SemaphoreType.DMA,
        ],
    )
    return kfn(src2d, dst2d, y)


def _tc1_body(x, w1, degp, y, xw, dis):
    mm = jnp.dot(x[...], w1[...], preferred_element_type=jnp.float32)
    deg = degp[0] + degp[1] + 1.0
    di = lax.rsqrt(deg)
    xw[...] = mm
    y[...] = mm * di
    dis[...] = di


def _tc1(x, w1, degp):
    grid = (N // RB,)
    return pl.pallas_call(
        _tc1_body,
        grid=grid,
        in_specs=[
            pl.BlockSpec((RB, D), lambda i: (i, 0)),
            pl.BlockSpec((D, H), lambda i: (0, 0)),
            pl.BlockSpec((2, RB, 1), lambda i: (0, i, 0)),
        ],
        out_specs=[
            pl.BlockSpec((RB, H), lambda i: (i, 0)),
            pl.BlockSpec((RB, H), lambda i: (i, 0)),
            pl.BlockSpec((RB, 1), lambda i: (i, 0)),
        ],
        out_shape=[
            jax.ShapeDtypeStruct((N, H), jnp.float32),
            jax.ShapeDtypeStruct((N, H), jnp.float32),
            jax.ShapeDtypeStruct((N, 1), jnp.float32),
        ],
    )(x, w1, degp)


def _tc2_body(zp0, zp1, xw, dis, b1, w2p, y2):
    di = dis[...]
    z = zp0[...] + zp1[...]
    h = jnp.maximum(z * di + xw[...] * (di * di) + b1[...], 0.0)
    mm = jnp.dot(h, w2p[...], preferred_element_type=jnp.float32)
    y2[...] = mm * di


def _tc2(zp0, zp1, xw, dis, b1, w2p, wo):
    grid = (N // RB,)
    return pl.pallas_call(
        _tc2_body,
        grid=grid,
        in_specs=[
            pl.BlockSpec((RB, H), lambda i: (i, 0)),
            pl.BlockSpec((RB, H), lambda i: (i, 0)),
            pl.BlockSpec((RB, H), lambda i: (i, 0)),
            pl.BlockSpec((RB, 1), lambda i: (i, 0)),
            pl.BlockSpec((1, H), lambda i: (0, 0)),
            pl.BlockSpec((H, wo), lambda i: (0, 0)),
        ],
        out_specs=pl.BlockSpec((RB, wo), lambda i: (i, 0)),
        out_shape=jax.ShapeDtypeStruct((N, wo), jnp.float32),
    )(zp0, zp1, xw, dis, b1, w2p)


def _agg2_kernel(ch2, src2d, dst2d, y2c0, y2c1, disp, b2p):
    def body(src2d, dst2d, y2c0, y2c1, disp, b2p, out,
             srcbuf, dstbuf, ybuf0, ybuf1, acc0, acc1, dbuf, bbuf,
             sbuf, obuf, shared):
        sid = lax.axis_index("s")

        pltpu.sync_copy(src2d.at[pl.ds(sid * ch2, ch2)], srcbuf)
        pltpu.sync_copy(dst2d.at[pl.ds(sid * ch2, ch2)], dstbuf)
        pltpu.sync_copy(y2c0, ybuf0)
        pltpu.sync_copy(y2c1, ybuf1)

        zeros = jnp.zeros((16,), jnp.float32)

        def zero_body(i, carry):
            acc0[pl.ds(i * 16, 16)] = zeros
            acc1[pl.ds(i * 16, 16)] = zeros
            return carry
        lax.fori_loop(0, NB // 16, zero_body, 0)

        @pl.when(sid == 0)
        def _():
            def seed_body(i, carry):
                acc0[pl.ds(i * 16, 16)] = ybuf0[pl.ds(i * 16, 16)]
                acc1[pl.ds(i * 16, 16)] = ybuf1[pl.ds(i * 16, 16)]
                return carry
            lax.fori_loop(0, N // 16, seed_body, 0)

        def edge_body(j, carry):
            for k in range(LANES // 16):
                s = srcbuf[j, pl.ds(k * 16, 16)]
                d = dstbuf[j, pl.ds(k * 16, 16)]
                v0 = plsc.load_gather(ybuf0, [s])
                v1 = plsc.load_gather(ybuf1, [s])
                plsc.addupdate_scatter(acc0, [d], v0)
                plsc.addupdate_scatter(acc1, [d], v1)
            return carry
        lax.fori_loop(0, ch2, edge_body, 0)

        pltpu.sync_copy(acc0, shared.at[sid, 0])
        pltpu.sync_copy(acc1, shared.at[sid, 1])
        plsc.subcore_barrier()

        pltpu.sync_copy(disp.at[pl.ds(sid * SR, SR)], dbuf)
        pltpu.sync_copy(b2p, bbuf)
        zi = jnp.zeros((16,), jnp.int32)
        b2_0 = plsc.load_gather(bbuf, [zi])
        b2_1 = plsc.load_gather(bbuf, [zi + 1])

        for col, bias in ((0, b2_0), (1, b2_1)):
            pltpu.sync_copy(shared.at[:, col, pl.ds(sid * SR, SR)], sbuf)

            def sum_body(c, carry):
                v = sbuf[0, pl.ds(c * 16, 16)]
                for r in range(1, NS):
                    v = v + sbuf[r, pl.ds(c * 16, 16)]
                obuf[pl.ds(c * 16, 16)] = v * dbuf[pl.ds(c * 16, 16)] + bias
                return carry
            lax.fori_loop(0, SR // 16, sum_body, 0)
            pltpu.sync_copy(obuf, out.at[col, pl.ds(sid * SR, SR)])

    kfn = pl.kernel(
        body,
        out_type=jax.ShapeDtypeStruct((2, NB), jnp.float32),
        mesh=plsc.VectorSubcoreMesh(
            core_axis_name="c", subcore_axis_name="s", num_cores=1),
        scratch_types=[
            pltpu.VMEM((ch2, LANES), jnp.int32),
            pltpu.VMEM((ch2, LANES), jnp.int32),
            pltpu.VMEM((N, ), jnp.float32),
            pltpu.VMEM((N, ), jnp.float32),
            pltpu.VMEM((NB,), jnp.float32),
            pltpu.VMEM((NB,), jnp.float32),
            pltpu.VMEM((SR,), jnp.float32),
            pltpu.VMEM((16,), jnp.float32),
            pltpu.VMEM((NS, SR), jnp.float32),
            pltpu.VMEM((SR,), jnp.float32),
            pltpu.VMEM_SHARED((NS, 2, NB), jnp.float32),
        ],
        compiler_params=pltpu.CompilerParams(needs_layout_passes=False),
    )
    return kfn(src2d, dst2d, y2c0, y2c1, disp, b2p)


@jax.jit
def kernel(x, edge_index, W1, b1, W2, b2):
    E = edge_index.shape[1]
    out_dim = W2.shape[1]
    WO = 16

    ch = -(-E // (NW * LANES))
    ch = ((ch + 7) // 8) * 8
    e_pad = NW * ch * LANES
    src = edge_index[0].astype(jnp.int32)
    dst = edge_index[1].astype(jnp.int32)
    pad = e_pad - E
    src = jnp.concatenate([src, jnp.zeros((pad,), jnp.int32)])
    dst = jnp.concatenate([dst, jnp.full((pad,), N, jnp.int32)])
    src2d = src.reshape(NW * ch, LANES)
    dst2d = dst.reshape(NW * ch, LANES)

    degp = _deg_hist(dst2d, ch)
    y, xw, dis = _tc1(x, W1, degp.reshape(2, NB, 1))
    zp0, zp1 = _agg_kernel(ch, H, src2d, dst2d, y)
    w2p = jnp.zeros((H, WO), jnp.float32).at[:, :out_dim].set(W2)
    y2 = _tc2(zp0, zp1, xw, dis, b1.reshape(1, H), w2p, WO)
    y2c0 = y2[:, 0].reshape(N)
    y2c1 = y2[:, 1].reshape(N)
    disp = jnp.zeros((NB,), jnp.float32).at[:N].set(dis[:, 0])
    b2p = jnp.zeros((16,), jnp.float32).at[:out_dim].set(b2)
    ch2 = NW * ch // NS
    outT = _agg2_kernel(ch2, src2d, dst2d, y2c0, y2c1, disp, b2p)
    return outT[:, :N].T
